# baseline (speedup 1.0000x reference)
import functools

import jax
import jax.numpy as jnp
from jax import lax
from jax.experimental import pallas as pl
from jax.experimental.pallas import tpu as pltpu
from jax.experimental.pallas import tpu_sc as plsc

NC = 2
NS = 16
NW = NC * NS
L = 16
SUB = 8
LANE = 128


@functools.lru_cache(maxsize=None)
def _build(b: int, s: int, d: int):
  bw = b // NW
  nbt = bw // LANE
  ndt = d // SUB

  mesh = plsc.VectorSubcoreMesh(
      core_axis_name="c", subcore_axis_name="s", num_cores=NC, num_subcores=NS
  )

  @functools.partial(
      pl.kernel,
      mesh=mesh,
      out_type=jax.ShapeDtypeStruct(
          (s, ndt, (b // LANE) * SUB, LANE), jnp.float32
      ),
      scratch_types=[
          pltpu.VMEM((2, bw), jnp.int32),
          pltpu.VMEM((2, bw, d), jnp.float32),
          pltpu.VMEM((2, ndt * nbt * SUB, LANE + 1), jnp.float32),
          [pltpu.SemaphoreType.DMA] * 2,
          [pltpu.SemaphoreType.DMA] * 2,
      ],
      compiler_params=pltpu.CompilerParams(
          use_tc_tiling_on_sc=False,
          needs_layout_passes=False,
          disable_bounds_checks=True,
      ),
  )
  def gather(tok_hbm, table_hbm, out_hbm, idx_v, rows_v, rowst_v, gsem, osem):
    wid = lax.axis_index("s") * NC + lax.axis_index("c")
    bb0 = wid * bw
    iota = lax.iota(jnp.int32, L)
    zeros = jnp.zeros((L,), jnp.int32)
    pitch = LANE + 1
    rowpats = [
        [((h * L + iota) // SUB) * (nbt * SUB)
         + bt * SUB
         + (h * L + iota) % SUB
         for h in range(d // L)]
        for bt in range(nbt)
    ]

    def front(g, bf):
      pltpu.sync_copy(tok_hbm.at[pl.ds(g * b + bb0, bw)], idx_v.at[bf])
      pltpu.async_copy(table_hbm.at[idx_v.at[bf]], rows_v.at[bf], gsem[bf])

    def wait_gather(bf):
      pltpu.make_async_copy(
          table_hbm.at[idx_v.at[bf]], rows_v.at[bf], gsem[bf]
      ).wait()

    UNROLL = 2

    def transpose(bf):
      dst = rowst_v.at[bf]
      for bt in range(nbt):

        @plsc.parallel_loop(0, LANE // UNROLL)
        def _bi(q):
          for u in range(UNROLL):
            bi = q * UNROLL + u
            t = bt * LANE + bi
            bvec = zeros + bi
            for h in range(d // L):
              x = rows_v[bf, t, pl.ds(h * L, L)]
              plsc.store_scatter(dst, [rowpats[bt][h], bvec], x)

    def write(g, bf):
      for dt in range(ndt):
        pltpu.async_copy(
            rowst_v.at[bf, pl.ds(dt * nbt * SUB, nbt * SUB), pl.ds(0, LANE)],
            out_hbm.at[g, dt, pl.ds(wid * nbt * SUB, nbt * SUB)],
            osem[bf],
        )

    def wait_write(bf):
      for dt in range(ndt):
        pltpu.make_async_copy(
            rowst_v.at[bf, pl.ds(dt * nbt * SUB, nbt * SUB), pl.ds(0, LANE)],
            out_hbm.at[0, 0, pl.ds(0, nbt * SUB)],
            osem[bf],
        ).wait()

    front(0, 0)
    front(1, 1)

    @pl.loop(0, s // 2)
    def _blk(i):
      for bf in range(2):
        g = 2 * i + bf
        wait_gather(bf)

        @pl.when(g >= 2)
        def _():
          wait_write(bf)

        transpose(bf)

        @pl.when(g + 2 < s)
        def _():
          front(g + 2, bf)

        write(g, bf)

    wait_write(0)
    wait_write(1)

  return gather


def kernel(token, embeddings):
  b, s = token.shape
  d = embeddings.shape[1]
  tokf = token.T.reshape(s * b).astype(jnp.int32)
  out = _build(b, s, d)(tokf, embeddings)
  out = out.reshape(s, d // 8, b // 128, 8, 128)
  return out.transpose(2, 4, 0, 1, 3).reshape(b, s, d)


# speedup vs baseline: 1.0847x; 1.0847x over previous
import functools

import jax
import jax.numpy as jnp
from jax import lax
from jax.experimental import pallas as pl
from jax.experimental.pallas import tpu as pltpu
from jax.experimental.pallas import tpu_sc as plsc

NC = 2
NS = 16
NW = NC * NS
L = 16
SUB = 8
LANE = 128


@functools.lru_cache(maxsize=None)
def _build(b: int, s: int, d: int):
  bw = b // NW
  nbt = bw // LANE
  ndt = d // SUB

  mesh = plsc.VectorSubcoreMesh(
      core_axis_name="c", subcore_axis_name="s", num_cores=NC, num_subcores=NS
  )

  @functools.partial(
      pl.kernel,
      mesh=mesh,
      out_type=jax.ShapeDtypeStruct(
          (s, ndt, (b // LANE) * SUB, LANE), jnp.float32
      ),
      scratch_types=[
          pltpu.VMEM((2, bw), jnp.int32),
          pltpu.VMEM((2, bw, d), jnp.float32),
          pltpu.VMEM((2, ndt * nbt * SUB, LANE + 1), jnp.float32),
          [pltpu.SemaphoreType.DMA] * 2,
          [pltpu.SemaphoreType.DMA] * 2,
      ],
      compiler_params=pltpu.CompilerParams(
          use_tc_tiling_on_sc=False,
          needs_layout_passes=False,
          disable_bounds_checks=True,
      ),
  )
  def gather(tok_hbm, table_hbm, out_hbm, idx_v, rows_v, rowst_v, gsem, osem):
    wid = lax.axis_index("s") * NC + lax.axis_index("c")
    bb0 = wid * bw
    iota = lax.iota(jnp.int32, L)
    zeros = jnp.zeros((L,), jnp.int32)
    pitch = LANE + 1
    rowpats = [
        [((h * L + iota) // SUB) * (nbt * SUB)
         + bt * SUB
         + (h * L + iota) % SUB
         for h in range(d // L)]
        for bt in range(nbt)
    ]

    def front(g, bf):
      pltpu.sync_copy(tok_hbm.at[pl.ds(g * b + bb0, bw)], idx_v.at[bf])
      pltpu.async_copy(table_hbm.at[idx_v.at[bf]], rows_v.at[bf], gsem[bf])

    def wait_gather(bf):
      pltpu.make_async_copy(
          table_hbm.at[idx_v.at[bf]], rows_v.at[bf], gsem[bf]
      ).wait()

    UNROLL = 1

    def transpose(bf):
      dst = rowst_v.at[bf]
      for bt in range(nbt):

        @plsc.parallel_loop(0, LANE // UNROLL, unroll=4)
        def _bi(q):
          for u in range(UNROLL):
            bi = q * UNROLL + u
            t = bt * LANE + bi
            bvec = zeros + bi
            for h in range(d // L):
              x = rows_v[bf, t, pl.ds(h * L, L)]
              plsc.store_scatter(dst, [rowpats[bt][h], bvec], x)

    def write(g, bf):
      for dt in range(ndt):
        pltpu.async_copy(
            rowst_v.at[bf, pl.ds(dt * nbt * SUB, nbt * SUB), pl.ds(0, LANE)],
            out_hbm.at[g, dt, pl.ds(wid * nbt * SUB, nbt * SUB)],
            osem[bf],
        )

    def wait_write(bf):
      for dt in range(ndt):
        pltpu.make_async_copy(
            rowst_v.at[bf, pl.ds(dt * nbt * SUB, nbt * SUB), pl.ds(0, LANE)],
            out_hbm.at[0, 0, pl.ds(0, nbt * SUB)],
            osem[bf],
        ).wait()

    front(0, 0)
    front(1, 1)

    @pl.loop(0, s // 2)
    def _blk(i):
      for bf in range(2):
        g = 2 * i + bf
        wait_gather(bf)

        @pl.when(g >= 2)
        def _():
          wait_write(bf)

        transpose(bf)

        @pl.when(g + 2 < s)
        def _():
          front(g + 2, bf)

        write(g, bf)

    wait_write(0)
    wait_write(1)

  return gather


def kernel(token, embeddings):
  b, s = token.shape
  d = embeddings.shape[1]
  tokf = token.T.reshape(s * b).astype(jnp.int32)
  out = _build(b, s, d)(tokf, embeddings)
  out = out.reshape(s, d // 8, b // 128, 8, 128)
  return out.transpose(2, 4, 0, 1, 3).reshape(b, s, d)
